# baseline (speedup 1.0000x reference)
import functools

import jax
import jax.numpy as jnp
from jax import lax
from jax.experimental import pallas as pl
from jax.experimental.pallas import tpu as pltpu
from jax.experimental.pallas import tpu_sc as plsc

NC = 2
NS = 16
NW = NC * NS

IDX_W = 128
ROWS_PER_CHUNK = 4


@functools.partial(jax.jit, static_argnames=("n_rows", "dim"))
def _sc_gather(tok, table, *, n_rows, dim):
    chunk = ROWS_PER_CHUNK * IDX_W
    rows_per_w = n_rows // NW
    chunks_per_w = rows_per_w // ROWS_PER_CHUNK

    @functools.partial(
        pl.kernel,
        mesh=plsc.VectorSubcoreMesh(core_axis_name="c", subcore_axis_name="s"),
        out_type=jax.ShapeDtypeStruct((n_rows * IDX_W, dim), jnp.float32),
        scratch_types=[
            pltpu.VMEM((ROWS_PER_CHUNK, IDX_W), jnp.int32),
            pltpu.VMEM((chunk, dim), jnp.float32),
            pltpu.SemaphoreType.DMA,
        ],
        compiler_params=pltpu.CompilerParams(use_tc_tiling_on_sc=False),
    )
    def k(tok_hbm, table_hbm, out_hbm, idx_v, rows_v, sem):
        wid = lax.axis_index("s") * NC + lax.axis_index("c")
        w_row0 = wid * rows_per_w

        def body(g, carry):
            r0 = w_row0 + g * ROWS_PER_CHUNK
            pltpu.sync_copy(tok_hbm.at[pl.ds(r0, ROWS_PER_CHUNK)], idx_v)
            cps = [
                pltpu.async_copy(
                    table_hbm.at[idx_v.at[j]],
                    rows_v.at[pl.ds(j * IDX_W, IDX_W)],
                    sem,
                )
                for j in range(ROWS_PER_CHUNK)
            ]
            for cp in cps:
                cp.wait()
            pltpu.sync_copy(rows_v, out_hbm.at[pl.ds(r0 * IDX_W, chunk)])
            return carry

        lax.fori_loop(0, chunks_per_w, body, 0)

    return k(tok, table)


def kernel(token_ids, embedding):
    b, s = token_ids.shape
    v, dim = embedding.shape
    tok = token_ids.reshape(-1, IDX_W).astype(jnp.int32)
    out = _sc_gather(tok, embedding, n_rows=tok.shape[0], dim=dim)
    return out.reshape(b, s, dim)


# speedup vs baseline: 1.0437x; 1.0437x over previous
import functools

import jax
import jax.numpy as jnp
from jax import lax
from jax.experimental import pallas as pl
from jax.experimental.pallas import tpu as pltpu
from jax.experimental.pallas import tpu_sc as plsc

NC = 2
NS = 16
NW = NC * NS

IDX_W = 128
ROWS_PER_CHUNK = 2
NBUF = 4


@functools.partial(jax.jit, static_argnames=("n_rows", "dim"))
def _sc_gather(tok, table, *, n_rows, dim):
    chunk = ROWS_PER_CHUNK * IDX_W
    rows_per_w = n_rows // NW
    chunks_per_w = rows_per_w // ROWS_PER_CHUNK
    steady = chunks_per_w - NBUF
    assert steady % NBUF == 0

    @functools.partial(
        pl.kernel,
        mesh=plsc.VectorSubcoreMesh(core_axis_name="c", subcore_axis_name="s"),
        out_type=jax.ShapeDtypeStruct((n_rows * IDX_W, dim), jnp.float32),
        scratch_types=[
            pltpu.VMEM((rows_per_w, IDX_W), jnp.int32),
            pltpu.VMEM((NBUF, chunk, dim), jnp.float32),
            [pltpu.SemaphoreType.DMA] * NBUF,
        ],
        compiler_params=pltpu.CompilerParams(use_tc_tiling_on_sc=False),
    )
    def k(tok_hbm, table_hbm, out_hbm, idx_all, rb, gsems):
        wid = lax.axis_index("s") * NC + lax.axis_index("c")
        w_row0 = wid * rows_per_w

        pltpu.sync_copy(tok_hbm.at[pl.ds(w_row0, rows_per_w)], idx_all)

        def fire(g, b):
            for j in range(ROWS_PER_CHUNK):
                pltpu.async_copy(
                    table_hbm.at[idx_all.at[g * ROWS_PER_CHUNK + j]],
                    rb.at[b].at[pl.ds(j * IDX_W, IDX_W)],
                    gsems[b],
                )

        def drain_store(g, b):
            for j in range(ROWS_PER_CHUNK):
                pltpu.make_async_copy(
                    table_hbm.at[idx_all.at[g * ROWS_PER_CHUNK + j]],
                    rb.at[b].at[pl.ds(j * IDX_W, IDX_W)],
                    gsems[b],
                ).wait()
            out0 = (w_row0 + g * ROWS_PER_CHUNK) * IDX_W
            pltpu.sync_copy(rb.at[b], out_hbm.at[pl.ds(out0, chunk)])

        for b in range(NBUF):
            fire(b, b)

        def body(o, carry):
            for b in range(NBUF):
                g = o * NBUF + b
                drain_store(g, b)
                fire(g + NBUF, b)
            return carry

        lax.fori_loop(0, steady // NBUF, body, 0)

        for b in range(NBUF):
            drain_store(steady + b, b)

    return k(tok, table)


def kernel(token_ids, embedding):
    b, s = token_ids.shape
    v, dim = embedding.shape
    tok = token_ids.reshape(-1, IDX_W).astype(jnp.int32)
    out = _sc_gather(tok, embedding, n_rows=tok.shape[0], dim=dim)
    return out.reshape(b, s, dim)


# speedup vs baseline: 1.2797x; 1.2261x over previous
import functools

import jax
import jax.numpy as jnp
from jax import lax
from jax.experimental import pallas as pl
from jax.experimental.pallas import tpu as pltpu
from jax.experimental.pallas import tpu_sc as plsc

NC = 2
NS = 16
NW = NC * NS

IDX_W = 128
ROWS_PER_CHUNK = 1
NBUF = 4


@functools.partial(jax.jit, static_argnames=("n_rows", "dim"))
def _sc_gather(tok, table, *, n_rows, dim):
    chunk = ROWS_PER_CHUNK * IDX_W
    rows_per_w = n_rows // NW
    chunks_per_w = rows_per_w // ROWS_PER_CHUNK
    steady = chunks_per_w - NBUF
    assert steady % NBUF == 0
    pad_dim = table.shape[-1]

    @functools.partial(
        pl.kernel,
        mesh=plsc.VectorSubcoreMesh(core_axis_name="c", subcore_axis_name="s"),
        out_type=jax.ShapeDtypeStruct((n_rows * IDX_W, pad_dim), jnp.float32),
        scratch_types=[
            pltpu.VMEM((rows_per_w, IDX_W), jnp.int32),
            pltpu.VMEM((NBUF, chunk, pad_dim), jnp.float32),
            [pltpu.SemaphoreType.DMA] * NBUF,
        ],
    )
    def k(tok_hbm, table_hbm, out_hbm, idx_all, rb, gsems):
        wid = lax.axis_index("s") * NC + lax.axis_index("c")
        w_row0 = wid * rows_per_w

        pltpu.sync_copy(tok_hbm.at[pl.ds(w_row0, rows_per_w)], idx_all)

        def fire(g, b):
            for j in range(ROWS_PER_CHUNK):
                pltpu.async_copy(
                    table_hbm.at[idx_all.at[g * ROWS_PER_CHUNK + j]],
                    rb.at[b].at[pl.ds(j * IDX_W, IDX_W)],
                    gsems[b],
                )

        def drain_store(g, b):
            for j in range(ROWS_PER_CHUNK):
                pltpu.make_async_copy(
                    table_hbm.at[idx_all.at[g * ROWS_PER_CHUNK + j]],
                    rb.at[b].at[pl.ds(j * IDX_W, IDX_W)],
                    gsems[b],
                ).wait()
            out0 = (w_row0 + g * ROWS_PER_CHUNK) * IDX_W
            pltpu.sync_copy(rb.at[b], out_hbm.at[pl.ds(out0, chunk)])

        for b in range(NBUF):
            fire(b, b)

        def body(o, carry):
            for b in range(NBUF):
                g = o * NBUF + b
                drain_store(g, b)
                fire(g + NBUF, b)
            return carry

        lax.fori_loop(0, steady // NBUF, body, 0)

        for b in range(NBUF):
            drain_store(steady + b, b)

    return k(tok, table)


def kernel(token_ids, embedding):
    b, s = token_ids.shape
    v, dim = embedding.shape
    tok = token_ids.reshape(-1, IDX_W).astype(jnp.int32)
    t_pad = jnp.pad(embedding, ((0, 0), (0, 128 - dim)))
    out = _sc_gather(tok, t_pad, n_rows=tok.shape[0], dim=dim)
    return out[:, :dim].reshape(b, s, dim)
